# Initial kernel scaffold; baseline (speedup 1.0000x reference)
import jax
import jax.numpy as jnp
from jax.experimental import pallas as pl


def kernel(x, w1, w2):
    raise NotImplementedError("write your pallas kernel here")


# speedup vs baseline: 1.1379x; 1.1379x over previous
import functools

import jax
import jax.numpy as jnp
from jax.experimental import pallas as pl
from jax.experimental.pallas import tpu as pltpu


def _se_body(x_ref, w1_ref, w2_ref, o_ref, *, inv_hw):
    x = x_ref[...]
    pooled = jnp.sum(x, axis=-1, keepdims=True, dtype=jnp.float32) * inv_hw
    h = jnp.dot(w1_ref[...], pooled, preferred_element_type=jnp.float32)
    h = jnp.maximum(h, 0.0)
    gate = jax.nn.sigmoid(
        jnp.dot(w2_ref[...], h, preferred_element_type=jnp.float32))
    o_ref[...] = x * gate.astype(o_ref.dtype)


def kernel(x, w1, w2):
    B, C, H, W = x.shape
    Cr = w1.shape[0]
    HW = H * W

    x_flat = x.reshape(B, C, HW)

    out_flat = pl.pallas_call(
        functools.partial(_se_body, inv_hw=1.0 / HW),
        out_shape=jax.ShapeDtypeStruct((B, C, HW), x.dtype),
        grid=(B,),
        in_specs=[
            pl.BlockSpec((None, C, HW), lambda b: (b, 0, 0)),
            pl.BlockSpec((Cr, C), lambda b: (0, 0)),
            pl.BlockSpec((C, Cr), lambda b: (0, 0)),
        ],
        out_specs=pl.BlockSpec((None, C, HW), lambda b: (b, 0, 0)),
        compiler_params=pltpu.CompilerParams(
            dimension_semantics=("parallel",),
            vmem_limit_bytes=64 * 1024 * 1024,
        ),
    )(x_flat, w1, w2)

    return out_flat.reshape(B, C, H, W)
